# baseline (speedup 1.0000x reference)
import jax
import jax.numpy as jnp
from jax.experimental import pallas as pl
from jax.experimental.pallas import tpu as pltpu

_N_USERS = 100000
_N_ITEMS = 1000000
_EMB = 64
_NTOT = _N_USERS + _N_ITEMS
_W = 16384
_IB = _N_USERS // _W
_R = _N_USERS - _IB * _W
_GRID = -(-_NTOT // _W)


def _body(u_ref, i_ref, o_ref, tail_ref):
    i = pl.program_id(0)

    @pl.when(i < _IB)
    def _():
        o_ref[...] = u_ref[...]

    @pl.when(i >= _IB)
    def _():
        blk = i_ref[...]
        head = jnp.where(i == _IB, u_ref[:, :_R], tail_ref[...])
        o_ref[...] = jnp.concatenate([head, blk[:, : _W - _R]], axis=1)
        tail_ref[...] = blk[:, _W - _R :]


def kernel(user_emb, item_emb):
    out_t = pl.pallas_call(
        _body,
        grid=(_GRID,),
        out_shape=jax.ShapeDtypeStruct((_EMB, _NTOT), jnp.float32),
        in_specs=[
            pl.BlockSpec((_EMB, _W), lambda i: (0, jnp.minimum(i, _IB))),
            pl.BlockSpec((_EMB, _W), lambda i: (0, jnp.maximum(i - _IB, 0))),
        ],
        out_specs=pl.BlockSpec((_EMB, _W), lambda i: (0, i)),
        scratch_shapes=[pltpu.VMEM((_EMB, _R), jnp.float32)],
    )(user_emb.T, item_emb.T)
    return out_t.T


# speedup vs baseline: 1.0412x; 1.0412x over previous
import jax
import jax.numpy as jnp
from jax.experimental import pallas as pl
from jax.experimental.pallas import tpu as pltpu

_N_USERS = 100000
_N_ITEMS = 1000000
_EMB = 64
_NTOT = _N_USERS + _N_ITEMS
_W = 33280
_IB = _N_USERS // _W
_R = _N_USERS - _IB * _W
_GRID = -(-_NTOT // _W)


def _body(u_ref, i_ref, o_ref, tail_ref):
    i = pl.program_id(0)

    @pl.when(i < _IB)
    def _():
        o_ref[...] = u_ref[...]

    @pl.when(i >= _IB)
    def _():
        blk = i_ref[...]
        head = jnp.where(i == _IB, u_ref[:, :_R], tail_ref[...])
        o_ref[...] = jnp.concatenate([head, blk[:, : _W - _R]], axis=1)
        tail_ref[...] = blk[:, _W - _R :]


def kernel(user_emb, item_emb):
    out_t = pl.pallas_call(
        _body,
        grid=(_GRID,),
        out_shape=jax.ShapeDtypeStruct((_EMB, _NTOT), jnp.float32),
        in_specs=[
            pl.BlockSpec((_EMB, _W), lambda i: (0, jnp.minimum(i, _IB))),
            pl.BlockSpec((_EMB, _W), lambda i: (0, jnp.maximum(i - _IB, 0))),
        ],
        out_specs=pl.BlockSpec((_EMB, _W), lambda i: (0, i)),
        scratch_shapes=[pltpu.VMEM((_EMB, _R), jnp.float32)],
        compiler_params=pltpu.CompilerParams(vmem_limit_bytes=128 * 1024 * 1024),
    )(user_emb.T, item_emb.T)
    return out_t.T


# speedup vs baseline: 1.0465x; 1.0051x over previous
import jax
import jax.numpy as jnp
from jax.experimental import pallas as pl
from jax.experimental.pallas import tpu as pltpu

_N_USERS = 100000
_N_ITEMS = 1000000
_EMB = 64
_NTOT = _N_USERS + _N_ITEMS
_W = 32768
_IB = _N_USERS // _W
_R = _N_USERS - _IB * _W
_GRID = -(-_NTOT // _W)


def _body(u_ref, i_ref, o_ref, tail_ref):
    i = pl.program_id(0)

    @pl.when(i < _IB)
    def _():
        o_ref[...] = u_ref[...]

    @pl.when(i >= _IB)
    def _():
        blk = i_ref[...]
        head = jnp.where(i == _IB, u_ref[:, :_R], tail_ref[...])
        o_ref[...] = jnp.concatenate([head, blk[:, : _W - _R]], axis=1)
        tail_ref[...] = blk[:, _W - _R :]


def kernel(user_emb, item_emb):
    out_t = pl.pallas_call(
        _body,
        grid=(_GRID,),
        out_shape=jax.ShapeDtypeStruct((_EMB, _NTOT), jnp.float32),
        in_specs=[
            pl.BlockSpec((_EMB, _W), lambda i: (0, jnp.minimum(i, _IB))),
            pl.BlockSpec((_EMB, _W), lambda i: (0, jnp.maximum(i - _IB, 0))),
        ],
        out_specs=pl.BlockSpec((_EMB, _W), lambda i: (0, i)),
        scratch_shapes=[pltpu.VMEM((_EMB, _R), jnp.float32)],
        compiler_params=pltpu.CompilerParams(vmem_limit_bytes=128 * 1024 * 1024),
    )(user_emb.T, item_emb.T)
    return out_t.T
